# Initial kernel scaffold; baseline (speedup 1.0000x reference)
import jax
import jax.numpy as jnp
from jax.experimental import pallas as pl


def kernel(x):
    raise NotImplementedError("write your pallas kernel here")


# speedup vs baseline: 1.2587x; 1.2587x over previous
import functools

import jax
import jax.numpy as jnp
from jax.experimental import pallas as pl
from jax.experimental.pallas import tpu as pltpu

_K = 384
_SEARCH_BITS = 31


def _kwta_body(x_ref, out_ref, gate_ref, *, k):
    bb, t, c = x_ref.shape
    s = jnp.sum(jnp.abs(x_ref[...]), axis=1) / jnp.float32(t)

    s_bits = jax.lax.bitcast_convert_type(s, jnp.int32)
    lo0 = jnp.zeros((bb, 1), jnp.int32)
    hi0 = jnp.full((bb, 1), jnp.int32(0x7F800000))

    def step(_, carry):
        lo, hi = carry
        mid = lo + ((hi - lo + 1) >> 1)
        cnt = jnp.sum((s_bits >= mid).astype(jnp.int32), axis=1,
                      keepdims=True)
        take = cnt >= k
        lo = jnp.where(take, mid, lo)
        hi = jnp.where(take, hi, mid - 1)
        return lo, hi

    lo, _ = jax.lax.fori_loop(0, _SEARCH_BITS, step, (lo0, hi0))
    thresh = jax.lax.bitcast_convert_type(lo, jnp.float32)

    gate = (s >= thresh).astype(jnp.float32)
    gate_ref[...] = gate[None]
    out_ref[...] = x_ref[...] * gate[:, None, :]


def kernel(x):
    b, t, c = x.shape
    bb = 4
    grid = (b // bb,)
    out_gated, gate = pl.pallas_call(
        functools.partial(_kwta_body, k=_K),
        grid=grid,
        in_specs=[pl.BlockSpec((bb, t, c), lambda i: (i, 0, 0))],
        out_specs=[
            pl.BlockSpec((bb, t, c), lambda i: (i, 0, 0)),
            pl.BlockSpec((1, bb, c), lambda i: (i, 0, 0)),
        ],
        out_shape=[
            jax.ShapeDtypeStruct((b, t, c), x.dtype),
            jax.ShapeDtypeStruct((b // bb, bb, c), x.dtype),
        ],
        compiler_params=pltpu.CompilerParams(
            dimension_semantics=("arbitrary",),
        ),
    )(x)
    return (out_gated, gate.reshape(b, c))
